# baseline (speedup 1.0000x reference)
import functools

import jax
import jax.numpy as jnp
from jax import lax
from jax.experimental import pallas as pl
from jax.experimental.pallas import tpu as pltpu
from jax.experimental.pallas import tpu_sc as plsc

B = 64
M = 20
T = 30
D = 128
NN = 5
BM = B * M
NW = 32
PAIRS_PER_W = BM // NW
CHUNK_PAIRS = 4
ROWS_PER_CHUNK = CHUNK_PAIRS * T
NCHUNKS = PAIRS_PER_W // CHUNK_PAIRS
EPW = PAIRS_PER_W * T
NV = D // 16


@functools.lru_cache(maxsize=1)
def _make_sc_pool():
    mesh = plsc.VectorSubcoreMesh(core_axis_name="c", subcore_axis_name="s")
    return pl.kernel(
        _sc_pool_body,
        mesh=mesh,
        out_type=jax.ShapeDtypeStruct((B, M, D), jnp.float32),
        scratch_types=[
            pltpu.VMEM((EPW,), jnp.int32),
            pltpu.VMEM((EPW,), jnp.int32),
            pltpu.VMEM((4, ROWS_PER_CHUNK, D), jnp.float32),
            pltpu.VMEM((2, M, D), jnp.float32),
            pltpu.SemaphoreType.DMA,
            pltpu.SemaphoreType.DMA,
            pltpu.SemaphoreType.DMA,
            pltpu.SemaphoreType.DMA,
        ],
    )


def _sc_pool_body(packed_hbm, word_hbm, out_hbm, raw_v, idx_v, rows_v,
                  acc_v, sem0, sem1, sem2, sem3):
    wid = lax.axis_index("s") * 2 + lax.axis_index("c")
    base_e = wid * EPW
    pltpu.sync_copy(packed_hbm.at[pl.ds(base_e, EPW)], raw_v)

    def clean_body(k, carry):
        v = raw_v[pl.ds(k * 16, 16)]
        idx_v[pl.ds(k * 16, 16)] = v & jnp.int32(0x1FFFF)
        return carry

    n_prime = (4 * ROWS_PER_CHUNK + 15) // 16
    lax.fori_loop(0, n_prime, clean_body, 0)

    def gather_ref(c):
        return word_hbm.at[idx_v.at[pl.ds(c * ROWS_PER_CHUNK,
                                          ROWS_PER_CHUNK)]]

    pltpu.async_copy(gather_ref(0), rows_v.at[0], sem0)
    pltpu.async_copy(gather_ref(1), rows_v.at[1], sem1)
    pltpu.async_copy(gather_ref(2), rows_v.at[2], sem2)
    pltpu.async_copy(gather_ref(3), rows_v.at[3], sem3)
    lax.fori_loop(n_prime, EPW // 16, clean_body, 0)

    def compute_chunk(c, buf):
        def pair_body(p, carry2):
            pair = c * CHUNK_PAIRS + p
            accs = [jnp.zeros((16,), jnp.float32) for _ in range(NV)]
            wsum = jnp.float32(0.0)
            scale = jnp.float32(1.0 / 32767.0)
            wv0 = lax.convert_element_type(
                lax.shift_right_logical(raw_v[pl.ds(pair * T, 16)],
                                        jnp.int32(17)),
                jnp.float32) * scale
            wv1 = lax.convert_element_type(
                lax.shift_right_logical(
                    raw_v[pl.ds(pair * T + (T - 16), 16)], jnp.int32(17)),
                jnp.float32) * scale
            for t in range(T):
                w = wv0[t] if t < 16 else wv1[t - (T - 16)]
                wsum = wsum + w
                for j in range(NV):
                    accs[j] = accs[j] + w * rows_v[buf, p * T + t,
                                                   pl.ds(j * 16, 16)]
            denom = jnp.full((16,), wsum, jnp.float32) + jnp.float32(1e-8)
            for j in range(NV):
                acc_v[pair // M, pair % M, pl.ds(j * 16, 16)] = (
                    accs[j] / denom)
            return carry2

        lax.fori_loop(0, CHUNK_PAIRS, pair_body, 0)

    def chunk_body(c, carry):
        def do(buf, sem):
            pltpu.make_async_copy(gather_ref(c), rows_v.at[buf], sem).wait()
            compute_chunk(c, buf)

            @pl.when(c + 4 < NCHUNKS)
            def _():
                pltpu.async_copy(gather_ref(c + 4), rows_v.at[buf], sem)

        @pl.when(c % 4 == 0)
        def _():
            do(0, sem0)

        @pl.when(c % 4 == 1)
        def _():
            do(1, sem1)

        @pl.when(c % 4 == 2)
        def _():
            do(2, sem2)

        @pl.when(c % 4 == 3)
        def _():
            do(3, sem3)

        return carry

    lax.fori_loop(0, NCHUNKS, chunk_body, 0)
    pltpu.sync_copy(acc_v, out_hbm.at[pl.ds(wid * 2, 2)])


def _tc_body(pooled_ref, wnews_ref, wl_ref, lb_ref, wr_ref, kw_ref, qw_ref,
             qb_ref, cand_ref, out_ref):
    f32 = jnp.float32
    pooled = pooled_ref[...]
    hist = jnp.tanh(lax.dot_general(pooled, wnews_ref[...],
                                    (((2,), (0,)), ((), ()))))
    hbar = jnp.mean(hist, axis=0)
    hbar_wl = lax.dot_general(hbar, wl_ref[...],
                              (((1,), (1,)), ((), ()))) + lb_ref[...]
    umask = (lax.broadcasted_iota(jnp.int32, (B, 1, 1), 0) < M).astype(f32)
    gcn = (lax.dot_general(hist, wr_ref[...], (((2,), (1,)), ((), ())))
           + hbar_wl[None, :, :] * umask)
    q = lax.dot_general(cand_ref[...], qw_ref[...],
                        (((2,), (1,)), ((), ()))) + qb_ref[...]
    c2 = lax.dot_general(q, kw_ref[...], (((2,), (0,)), ((), ())))
    scores = lax.dot_general(gcn, c2, (((2,), (2,)), ((0,), (0,))))
    scores = scores * f32(1.0 / (128.0 ** 0.5))
    mx = jnp.max(scores, axis=1, keepdims=True)
    e = jnp.exp(scores - mx)
    alpha = e / jnp.sum(e, axis=1, keepdims=True)
    out_ref[...] = lax.dot_general(alpha, gcn,
                                   (((1,), (1,)), ((0,), (0,))))


_tc_dense = pl.pallas_call(
    _tc_body,
    out_shape=jax.ShapeDtypeStruct((B, NN, D), jnp.float32),
)


def kernel(user_title_text, user_title_mask, user_title_entity,
           user_content_text, user_content_mask, user_content_entity,
           category, user_category, user_subCategory, user_history_mask,
           user_history_graph, user_history_category_mask,
           user_history_category_indices, user_embedding,
           candidate_news_representation, user_freshness,
           user_user_topic_lifetime, word_emb, category_emb, W_news,
           user_node_embedding, sage_lin_l_W, sage_lin_l_b, sage_lin_r_W,
           K_W, Q_W, Q_b):
    text = user_title_text.astype(jnp.int32)
    mq = jnp.minimum(
        (user_title_mask.astype(jnp.float32) * 32767.0).astype(jnp.int32),
        32767)
    packed = (text | (mq << 17)).reshape(BM * T)
    pooled = _make_sc_pool()(packed, word_emb)
    return _tc_dense(pooled, W_news, sage_lin_l_W,
                     sage_lin_l_b.reshape(1, D), sage_lin_r_W, K_W, Q_W,
                     Q_b.reshape(1, D), candidate_news_representation)


# speedup vs baseline: 1.0141x; 1.0141x over previous
import functools

import jax
import jax.numpy as jnp
from jax import lax
from jax.experimental import pallas as pl
from jax.experimental.pallas import tpu as pltpu
from jax.experimental.pallas import tpu_sc as plsc

B = 64
M = 20
T = 30
D = 128
NN = 5
BM = B * M
NW = 32
PAIRS_PER_W = BM // NW
CHUNK_PAIRS = 4
ROWS_PER_CHUNK = CHUNK_PAIRS * T
NCHUNKS = PAIRS_PER_W // CHUNK_PAIRS
EPW = PAIRS_PER_W * T
NV = D // 16


@functools.lru_cache(maxsize=1)
def _make_sc_pool():
    mesh = plsc.VectorSubcoreMesh(core_axis_name="c", subcore_axis_name="s")
    return pl.kernel(
        _sc_pool_body,
        mesh=mesh,
        out_type=jax.ShapeDtypeStruct((B, M, D), jnp.float32),
        scratch_types=[
            pltpu.VMEM((EPW,), jnp.int32),
            pltpu.VMEM((EPW,), jnp.int32),
            pltpu.VMEM((3, ROWS_PER_CHUNK, D), jnp.float32),
            pltpu.VMEM((2, M, D), jnp.float32),
            pltpu.SemaphoreType.DMA,
            pltpu.SemaphoreType.DMA,
            pltpu.SemaphoreType.DMA,
        ],
    )


def _sc_pool_body(packed_hbm, word_hbm, out_hbm, raw_v, idx_v, rows_v,
                  acc_v, sem0, sem1, sem2):
    wid = lax.axis_index("s") * 2 + lax.axis_index("c")
    base_e = wid * EPW
    pltpu.sync_copy(packed_hbm.at[pl.ds(base_e, EPW)], raw_v)

    def clean_body(k, carry):
        v = raw_v[pl.ds(k * 16, 16)]
        idx_v[pl.ds(k * 16, 16)] = v & jnp.int32(0x1FFFF)
        return carry

    n_prime = (3 * ROWS_PER_CHUNK + 15) // 16
    lax.fori_loop(0, n_prime, clean_body, 0)

    def gather_ref(c):
        return word_hbm.at[idx_v.at[pl.ds(c * ROWS_PER_CHUNK,
                                          ROWS_PER_CHUNK)]]

    pltpu.async_copy(gather_ref(0), rows_v.at[0], sem0)
    pltpu.async_copy(gather_ref(1), rows_v.at[1], sem1)
    pltpu.async_copy(gather_ref(2), rows_v.at[2], sem2)
    lax.fori_loop(n_prime, EPW // 16, clean_body, 0)

    def compute_chunk(c, buf):
        def pair_body(p, carry2):
            pair = c * CHUNK_PAIRS + p
            accs = [jnp.zeros((16,), jnp.float32) for _ in range(NV)]
            wsum = jnp.float32(0.0)
            scale = jnp.float32(1.0 / 32767.0)
            wv0 = lax.convert_element_type(
                lax.shift_right_logical(raw_v[pl.ds(pair * T, 16)],
                                        jnp.int32(17)),
                jnp.float32) * scale
            wv1 = lax.convert_element_type(
                lax.shift_right_logical(
                    raw_v[pl.ds(pair * T + (T - 16), 16)], jnp.int32(17)),
                jnp.float32) * scale
            for t in range(T):
                w = wv0[t] if t < 16 else wv1[t - (T - 16)]
                wsum = wsum + w
                for j in range(NV):
                    accs[j] = accs[j] + w * rows_v[buf, p * T + t,
                                                   pl.ds(j * 16, 16)]
            denom = jnp.full((16,), wsum, jnp.float32) + jnp.float32(1e-8)
            for j in range(NV):
                acc_v[pair // M, pair % M, pl.ds(j * 16, 16)] = (
                    accs[j] / denom)
            return carry2

        lax.fori_loop(0, CHUNK_PAIRS, pair_body, 0)

    def chunk_body(c, carry):
        def do(buf, sem):
            pltpu.make_async_copy(gather_ref(c), rows_v.at[buf], sem).wait()
            compute_chunk(c, buf)

            @pl.when(c + 3 < NCHUNKS)
            def _():
                pltpu.async_copy(gather_ref(c + 3), rows_v.at[buf], sem)

        @pl.when(c % 3 == 0)
        def _():
            do(0, sem0)

        @pl.when(c % 3 == 1)
        def _():
            do(1, sem1)

        @pl.when(c % 3 == 2)
        def _():
            do(2, sem2)

        return carry

    lax.fori_loop(0, NCHUNKS, chunk_body, 0)
    pltpu.sync_copy(acc_v, out_hbm.at[pl.ds(wid * 2, 2)])


def _tc_body(pooled_ref, wnews_ref, wl_ref, lb_ref, wr_ref, kw_ref, qw_ref,
             qb_ref, cand_ref, out_ref):
    f32 = jnp.float32
    pooled = pooled_ref[...]
    hist = jnp.tanh(lax.dot_general(pooled, wnews_ref[...],
                                    (((2,), (0,)), ((), ()))))
    hbar = jnp.mean(hist, axis=0)
    hbar_wl = lax.dot_general(hbar, wl_ref[...],
                              (((1,), (1,)), ((), ()))) + lb_ref[...]
    umask = (lax.broadcasted_iota(jnp.int32, (B, 1, 1), 0) < M).astype(f32)
    gcn = (lax.dot_general(hist, wr_ref[...], (((2,), (1,)), ((), ())))
           + hbar_wl[None, :, :] * umask)
    q = lax.dot_general(cand_ref[...], qw_ref[...],
                        (((2,), (1,)), ((), ()))) + qb_ref[...]
    c2 = lax.dot_general(q, kw_ref[...], (((2,), (0,)), ((), ())))
    scores = lax.dot_general(gcn, c2, (((2,), (2,)), ((0,), (0,))))
    scores = scores * f32(1.0 / (128.0 ** 0.5))
    mx = jnp.max(scores, axis=1, keepdims=True)
    e = jnp.exp(scores - mx)
    alpha = e / jnp.sum(e, axis=1, keepdims=True)
    out_ref[...] = lax.dot_general(alpha, gcn,
                                   (((1,), (1,)), ((0,), (0,))))


_tc_dense = pl.pallas_call(
    _tc_body,
    out_shape=jax.ShapeDtypeStruct((B, NN, D), jnp.float32),
)


def kernel(user_title_text, user_title_mask, user_title_entity,
           user_content_text, user_content_mask, user_content_entity,
           category, user_category, user_subCategory, user_history_mask,
           user_history_graph, user_history_category_mask,
           user_history_category_indices, user_embedding,
           candidate_news_representation, user_freshness,
           user_user_topic_lifetime, word_emb, category_emb, W_news,
           user_node_embedding, sage_lin_l_W, sage_lin_l_b, sage_lin_r_W,
           K_W, Q_W, Q_b):
    text = user_title_text.astype(jnp.int32)
    mq = jnp.minimum(
        (user_title_mask.astype(jnp.float32) * 32767.0).astype(jnp.int32),
        32767)
    packed = (text | (mq << 17)).reshape(BM * T)
    pooled = _make_sc_pool()(packed, word_emb)
    return _tc_dense(pooled, W_news, sage_lin_l_W,
                     sage_lin_l_b.reshape(1, D), sage_lin_r_W, K_W, Q_W,
                     Q_b.reshape(1, D), candidate_news_representation)
